# baseline (speedup 1.0000x reference)
import functools

import jax
import jax.numpy as jnp
from jax import lax
from jax.experimental import pallas as pl
from jax.experimental.pallas import tpu as pltpu
from jax.experimental.pallas import tpu_sc as plsc

VOCAB = 100000
WORD_DIM = 300
CHAR_DIM = 100
EXT_DIM = WORD_DIM + CHAR_DIM
BATCH = 1024
MAX_SEQ = 200
NTOK = BATCH * MAX_SEQ
_LAST = EXT_DIM - 3 * 128

_ROWS_PER_BLOCK = 1000


def _tables_body(w_ref, c_ref, t0_ref, t1_ref, t2_ref, t3_ref):
    w = w_ref[...]
    c = c_ref[...]
    t0_ref[...] = w[:, 0:128]
    t1_ref[...] = w[:, 128:256]
    t2_ref[...] = jnp.concatenate([w[:, 256:WORD_DIM], c[:, 0 : 128 - 44]], axis=-1)
    t3_ref[...] = jnp.concatenate(
        [c[:, 128 - 44 : CHAR_DIM], jnp.zeros((_ROWS_PER_BLOCK, 128 - _LAST), jnp.float32)],
        axis=-1,
    )


def _build_tables(W_word, char_emb_tensor):
    grid = (VOCAB // _ROWS_PER_BLOCK,)
    col_tab = jax.ShapeDtypeStruct((VOCAB, 128), jnp.float32)
    col_spec = pl.BlockSpec((_ROWS_PER_BLOCK, 128), lambda i: (i, 0))
    return pl.pallas_call(
        _tables_body,
        grid=grid,
        in_specs=[
            pl.BlockSpec((_ROWS_PER_BLOCK, WORD_DIM), lambda i: (i, 0)),
            pl.BlockSpec((_ROWS_PER_BLOCK, CHAR_DIM), lambda i: (i, 0)),
        ],
        out_specs=[col_spec, col_spec, col_spec, col_spec],
        out_shape=[col_tab, col_tab, col_tab, col_tab],
    )(W_word, char_emb_tensor)


def _ext_body(w_ref, c_ref, o_ref):
    o_ref[...] = jnp.concatenate([w_ref[...], c_ref[...]], axis=-1)


def _build_ext(W_word, char_emb_tensor):
    grid = (VOCAB // _ROWS_PER_BLOCK,)
    return pl.pallas_call(
        _ext_body,
        grid=grid,
        in_specs=[
            pl.BlockSpec((_ROWS_PER_BLOCK, WORD_DIM), lambda i: (i, 0)),
            pl.BlockSpec((_ROWS_PER_BLOCK, CHAR_DIM), lambda i: (i, 0)),
        ],
        out_specs=pl.BlockSpec((_ROWS_PER_BLOCK, EXT_DIM), lambda i: (i, 0)),
        out_shape=jax.ShapeDtypeStruct((VOCAB, EXT_DIM), jnp.float32),
    )(W_word, char_emb_tensor)


_NW = 32
_ROWS_PER_W = BATCH // _NW
_PER_W = NTOK // _NW
_CH = 40
_CPR = MAX_SEQ // _CH


def _make_sc_gather():
    mesh = plsc.VectorSubcoreMesh(core_axis_name="c", subcore_axis_name="s")

    @functools.partial(
        pl.kernel,
        mesh=mesh,
        compiler_params=pltpu.CompilerParams(use_tc_tiling_on_sc=False),
        out_type=jax.ShapeDtypeStruct((BATCH, MAX_SEQ, EXT_DIM), jnp.float32),
        scratch_types=[
            pltpu.VMEM((_PER_W,), jnp.int32),
            pltpu.VMEM((MAX_SEQ, 128), jnp.float32),
            pltpu.VMEM((MAX_SEQ, 128), jnp.float32),
            pltpu.VMEM((MAX_SEQ, 128), jnp.float32),
            pltpu.VMEM((MAX_SEQ, 128), jnp.float32),
            pltpu.SemaphoreType.DMA,
        ],
    )
    def gather_k(t0, t1, t2, t3, idx_hbm, out_hbm, idx_v, s0, s1, s2, s3, sem):
        tabs = (t0, t1, t2, t3)
        stages = (s0, s1, s2, s3)
        wid = lax.axis_index("s") * 2 + lax.axis_index("c")
        base = wid * _PER_W

        pltpu.sync_copy(idx_hbm.at[pl.ds(base, _PER_W)], idx_v)

        def body(b, carry):
            copies = []
            for s in range(_CPR):
                idx_sl = idx_v.at[pl.ds(b * MAX_SEQ + s * _CH, _CH)]
                for k in range(4):
                    copies.append(
                        pltpu.async_copy(
                            tabs[k].at[idx_sl],
                            stages[k].at[pl.ds(s * _CH, _CH)],
                            sem,
                        )
                    )
            for c in copies:
                c.wait()
            row = out_hbm.at[wid * _ROWS_PER_W + b]
            for k in range(3):
                pltpu.sync_copy(stages[k], row.at[:, pl.ds(k * 128, 128)])
            pltpu.sync_copy(s3.at[:, pl.ds(0, _LAST)], row.at[:, pl.ds(384, _LAST)])
            return carry

        lax.fori_loop(0, _ROWS_PER_W, body, 0)

    return gather_k


_sc_gather = _make_sc_gather()


def kernel(word_id_seq, W_word, char_emb_tensor):
    t0, t1, t2, t3 = _build_tables(W_word, char_emb_tensor)
    flat_ids = word_id_seq.reshape(NTOK).astype(jnp.int32)
    embedded_seq = _sc_gather(t0, t1, t2, t3, flat_ids)
    ext = _build_ext(W_word, char_emb_tensor)
    return (embedded_seq, ext, word_id_seq)


# speedup vs baseline: 1.0566x; 1.0566x over previous
import functools

import jax
import jax.numpy as jnp
from jax import lax
from jax.experimental import pallas as pl
from jax.experimental.pallas import tpu as pltpu
from jax.experimental.pallas import tpu_sc as plsc

VOCAB = 100000
WORD_DIM = 300
CHAR_DIM = 100
EXT_DIM = WORD_DIM + CHAR_DIM
BATCH = 1024
MAX_SEQ = 200
NTOK = BATCH * MAX_SEQ
_LAST = EXT_DIM - 3 * 128

_ROWS_PER_BLOCK = 1000


def _concat_body(w_ref, c_ref, o_ref, t0_ref, t1_ref, t2_ref, t3_ref):
    w = w_ref[...]
    c = c_ref[...]
    o_ref[...] = jnp.concatenate([w, c], axis=-1)
    t0_ref[...] = w[:, 0:128]
    t1_ref[...] = w[:, 128:256]
    t2_ref[...] = jnp.concatenate([w[:, 256:WORD_DIM], c[:, 0 : 128 - 44]], axis=-1)
    t3_ref[...] = jnp.concatenate(
        [c[:, 128 - 44 : CHAR_DIM], jnp.zeros((_ROWS_PER_BLOCK, 128 - _LAST), jnp.float32)],
        axis=-1,
    )


def _build_extended(W_word, char_emb_tensor):
    grid = (VOCAB // _ROWS_PER_BLOCK,)
    col_tab = jax.ShapeDtypeStruct((VOCAB, 128), jnp.float32)
    col_spec = pl.BlockSpec((_ROWS_PER_BLOCK, 128), lambda i: (i, 0))
    return pl.pallas_call(
        _concat_body,
        grid=grid,
        in_specs=[
            pl.BlockSpec((_ROWS_PER_BLOCK, WORD_DIM), lambda i: (i, 0)),
            pl.BlockSpec((_ROWS_PER_BLOCK, CHAR_DIM), lambda i: (i, 0)),
        ],
        out_specs=[
            pl.BlockSpec((_ROWS_PER_BLOCK, EXT_DIM), lambda i: (i, 0)),
            col_spec,
            col_spec,
            col_spec,
            col_spec,
        ],
        out_shape=[
            jax.ShapeDtypeStruct((VOCAB, EXT_DIM), jnp.float32),
            col_tab,
            col_tab,
            col_tab,
            col_tab,
        ],
    )(W_word, char_emb_tensor)


_NW = 32
_ROWS_PER_W = BATCH // _NW
_PER_W = NTOK // _NW
_CH = 40
_CPR = MAX_SEQ // _CH


def _make_sc_gather():
    mesh = plsc.VectorSubcoreMesh(core_axis_name="c", subcore_axis_name="s")

    @functools.partial(
        pl.kernel,
        mesh=mesh,
        compiler_params=pltpu.CompilerParams(use_tc_tiling_on_sc=False),
        out_type=jax.ShapeDtypeStruct((BATCH, MAX_SEQ, EXT_DIM), jnp.float32),
        scratch_types=[
            pltpu.VMEM((_PER_W,), jnp.int32),
            pltpu.VMEM((MAX_SEQ, 128), jnp.float32),
            pltpu.VMEM((MAX_SEQ, 128), jnp.float32),
            pltpu.VMEM((MAX_SEQ, 128), jnp.float32),
            pltpu.VMEM((MAX_SEQ, 128), jnp.float32),
            pltpu.SemaphoreType.DMA,
            pltpu.SemaphoreType.DMA,
        ],
    )
    def gather_k(t0, t1, t2, t3, idx_hbm, out_hbm, idx_v, s0, s1, s2, s3, sem, sem_wb):
        tabs = (t0, t1, t2, t3)
        stages = (s0, s1, s2, s3)
        wid = lax.axis_index("s") * 2 + lax.axis_index("c")
        base = wid * _PER_W

        pltpu.sync_copy(idx_hbm.at[pl.ds(base, _PER_W)], idx_v)

        def body(b, carry):
            copies = []
            for s in range(_CPR):
                idx_sl = idx_v.at[pl.ds(b * MAX_SEQ + s * _CH, _CH)]
                for k in range(4):
                    copies.append(
                        pltpu.async_copy(
                            tabs[k].at[idx_sl],
                            stages[k].at[pl.ds(s * _CH, _CH)],
                            sem,
                        )
                    )
            for c in copies:
                c.wait()
            row = out_hbm.at[wid * _ROWS_PER_W + b]
            wb = [
                pltpu.async_copy(stages[k], row.at[:, pl.ds(k * 128, 128)], sem_wb)
                for k in range(3)
            ]
            wb.append(
                pltpu.async_copy(
                    s3.at[:, pl.ds(0, _LAST)], row.at[:, pl.ds(384, _LAST)], sem_wb
                )
            )
            for c in wb:
                c.wait()
            return carry

        lax.fori_loop(0, _ROWS_PER_W, body, 0)

    return gather_k


_sc_gather = _make_sc_gather()


def kernel(word_id_seq, W_word, char_emb_tensor):
    ext, t0, t1, t2, t3 = _build_extended(W_word, char_emb_tensor)
    flat_ids = word_id_seq.reshape(NTOK).astype(jnp.int32)
    embedded_seq = _sc_gather(t0, t1, t2, t3, flat_ids)
    return (embedded_seq, ext, word_id_seq)
